# baseline (speedup 1.0000x reference)
import functools

import jax
import jax.numpy as jnp
from jax import lax
from jax.experimental import pallas as pl
from jax.experimental.pallas import tpu as pltpu
from jax.experimental.pallas import tpu_sc as plsc

B, C, H, W = 8, 320, 48, 48
HW = H * W
GQ = 3

NC, NS = 2, 16
NW = NC * NS
CB = 8
UNITS = B * (C // CB)
UPW = UNITS // NW
NB = 5


def _gain_body(n_ref, g_ref, o_ref):
    g = jnp.abs(g_ref[0])
    pad = jnp.zeros((1, GQ * 128 - C), jnp.float32)
    rows = jnp.concatenate([g, pad], axis=1).reshape(GQ, 1, 128)
    o_ref[...] = jnp.pad(rows, ((0, 0), (0, 7), (0, 0)))


def _gain_table(n, gain_matrix):
    g3 = gain_matrix.reshape(B, 1, C)
    return pl.pallas_call(
        _gain_body,
        grid_spec=pltpu.PrefetchScalarGridSpec(
            num_scalar_prefetch=1,
            grid=(B,),
            in_specs=[pl.BlockSpec((1, 1, C), lambda b, n_ref: (n_ref[b], 0, 0))],
            out_specs=pl.BlockSpec((GQ, 8, 128), lambda b, n_ref: (b, 0, 0)),
        ),
        out_shape=jax.ShapeDtypeStruct((B * GQ, 8, 128), jnp.float32),
    )(n.astype(jnp.int32), g3)


def _sc_body(g_hbm, x_hbm, o_hbm, g_v, ibuf, isem, osem):
    wid = lax.axis_index("s") * NC + lax.axis_index("c")
    base = wid * UPW
    pltpu.sync_copy(g_hbm, g_v)

    def unit_bc(u):
        bb = u // (C // CB)
        cc0 = lax.rem(u, C // CB) * CB
        return bb, cc0

    def start_in(u, k):
        bb, cc0 = unit_bc(u)
        pltpu.make_async_copy(
            x_hbm.at[bb, pl.ds(cc0, CB), :], ibuf.at[k], isem.at[k]
        ).start()

    for k in range(NB):
        start_in(base + k, k)

    def group(g, _):
        for k in range(NB):
            u = base + g * NB + k
            bb, cc0 = unit_bc(u)
            pltpu.make_async_copy(
                x_hbm.at[bb, pl.ds(cc0, CB), :], ibuf.at[k], isem.at[k]
            ).wait()

            for ch in range(CB):
                cc = cc0 + ch
                q = bb * GQ + cc // 128
                cl = lax.rem(cc, 128)
                gv = plsc.load_gather(
                    g_v,
                    [
                        jnp.full((16,), q, jnp.int32),
                        jnp.zeros((16,), jnp.int32),
                        jnp.broadcast_to(cl.astype(jnp.int32), (16,)),
                    ],
                )

                def row(t, _):
                    col = t * 128
                    for j in range(8):
                        sl = pl.ds(col + j * 16, 16)
                        ibuf[k, ch, sl] = ibuf[k, ch, sl] * gv
                    return 0

                lax.fori_loop(0, HW // 128, row, 0)

            out_copy = pltpu.make_async_copy(
                ibuf.at[k], o_hbm.at[bb, pl.ds(cc0, CB), :], osem.at[k]
            )
            out_copy.start()

            @pl.when(g + 1 < UPW // NB)
            def _():
                pltpu.make_async_copy(
                    ibuf.at[k], o_hbm.at[bb, pl.ds(cc0, CB), :], osem.at[k]
                ).wait()
                start_in(u + NB, k)

        return 0

    lax.fori_loop(0, UPW // NB, group, 0)

    for k in range(NB):
        u = base + (UPW // NB - 1) * NB + k
        bb, cc0 = unit_bc(u)
        pltpu.make_async_copy(
            ibuf.at[k], o_hbm.at[bb, pl.ds(cc0, CB), :], osem.at[k]
        ).wait()


def kernel(x, n, gain_matrix):
    x3 = x.reshape(B, C, HW)
    gained = _gain_table(n, gain_matrix)
    scale = functools.partial(
        pl.kernel,
        out_type=jax.ShapeDtypeStruct((B, C, HW), jnp.float32),
        mesh=plsc.VectorSubcoreMesh(core_axis_name="c", subcore_axis_name="s"),
        scratch_types=[
            pltpu.VMEM((B * GQ, 8, 128), jnp.float32),
            pltpu.VMEM((NB, CB, HW), jnp.float32),
            pltpu.SemaphoreType.DMA((NB,)),
            pltpu.SemaphoreType.DMA((NB,)),
        ],
        compiler_params=pltpu.CompilerParams(
            use_tc_tiling_on_sc=True, needs_layout_passes=False
        ),
    )(_sc_body)
    out = scale(gained, x3)
    return out.reshape(B, C, H, W)


# speedup vs baseline: 1.0413x; 1.0413x over previous
import functools

import jax
import jax.numpy as jnp
from jax import lax
from jax.experimental import pallas as pl
from jax.experimental.pallas import tpu as pltpu
from jax.experimental.pallas import tpu_sc as plsc

B, C, H, W = 8, 320, 48, 48
HW = H * W

NC, NS = 2, 16
NW = NC * NS
CB = 8
UNITS = B * (C // CB)
UPW = UNITS // NW
NB = 5


def _gain_body(n_ref, gt_ref, o_ref):
    b = pl.program_id(0)
    idx = n_ref[b]
    rows = jnp.abs(gt_ref[...])
    onehot = (
        jax.lax.broadcasted_iota(jnp.int32, (1, 8), 1) == idx
    ).astype(jnp.float32)
    g_col = jnp.sum(rows * onehot, axis=1, keepdims=True)
    o_ref[...] = jnp.broadcast_to(g_col, (C, 128))


def _gain_table(n, gain_matrix):
    gt = gain_matrix.T
    return pl.pallas_call(
        _gain_body,
        grid_spec=pltpu.PrefetchScalarGridSpec(
            num_scalar_prefetch=1,
            grid=(B,),
            in_specs=[pl.BlockSpec((C, 8), lambda b, n_ref: (0, 0))],
            out_specs=pl.BlockSpec((C, 128), lambda b, n_ref: (b, 0)),
        ),
        out_shape=jax.ShapeDtypeStruct((B * C, 128), jnp.float32),
    )(n.astype(jnp.int32), gt)


def _sc_body(g_hbm, x_hbm, o_hbm, gbuf, ibuf, gsem, isem, osem):
    wid = lax.axis_index("s") * NC + lax.axis_index("c")
    base = wid * UPW

    def unit_bc(u):
        bb = u // (C // CB)
        cc0 = lax.rem(u, C // CB) * CB
        return bb, cc0

    def start_in(u, k):
        bb, cc0 = unit_bc(u)
        pltpu.make_async_copy(
            x_hbm.at[bb, pl.ds(cc0, CB), :], ibuf.at[k], isem.at[k]
        ).start()
        pltpu.make_async_copy(
            g_hbm.at[pl.ds(bb * C + cc0, CB), :], gbuf.at[k], gsem.at[k]
        ).start()

    for k in range(NB):
        start_in(base + k, k)

    def group(g, _):
        for k in range(NB):
            u = base + g * NB + k
            bb, cc0 = unit_bc(u)
            pltpu.make_async_copy(
                x_hbm.at[bb, pl.ds(cc0, CB), :], ibuf.at[k], isem.at[k]
            ).wait()
            pltpu.make_async_copy(
                g_hbm.at[pl.ds(bb * C + cc0, CB), :], gbuf.at[k], gsem.at[k]
            ).wait()

            for ch in range(CB):
                gv = gbuf[k, ch, pl.ds(0, 16)]

                def row(t, _):
                    col = t * 128
                    for j in range(8):
                        sl = pl.ds(col + j * 16, 16)
                        ibuf[k, ch, sl] = ibuf[k, ch, sl] * gv
                    return 0

                lax.fori_loop(0, HW // 128, row, 0)

            pltpu.make_async_copy(
                ibuf.at[k], o_hbm.at[bb, pl.ds(cc0, CB), :], osem.at[k]
            ).start()

            @pl.when(g + 1 < UPW // NB)
            def _():
                pltpu.make_async_copy(
                    ibuf.at[k], o_hbm.at[bb, pl.ds(cc0, CB), :], osem.at[k]
                ).wait()
                start_in(u + NB, k)

        return 0

    lax.fori_loop(0, UPW // NB, group, 0)

    for k in range(NB):
        u = base + (UPW // NB - 1) * NB + k
        bb, cc0 = unit_bc(u)
        pltpu.make_async_copy(
            ibuf.at[k], o_hbm.at[bb, pl.ds(cc0, CB), :], osem.at[k]
        ).wait()


def kernel(x, n, gain_matrix):
    x3 = x.reshape(B, C, HW)
    gained = _gain_table(n, gain_matrix)
    scale = functools.partial(
        pl.kernel,
        out_type=jax.ShapeDtypeStruct((B, C, HW), jnp.float32),
        mesh=plsc.VectorSubcoreMesh(core_axis_name="c", subcore_axis_name="s"),
        scratch_types=[
            pltpu.VMEM((NB, CB, 128), jnp.float32),
            pltpu.VMEM((NB, CB, HW), jnp.float32),
            pltpu.SemaphoreType.DMA((NB,)),
            pltpu.SemaphoreType.DMA((NB,)),
            pltpu.SemaphoreType.DMA((NB,)),
        ],
        compiler_params=pltpu.CompilerParams(use_tc_tiling_on_sc=True),
    )(_sc_body)
    out = scale(gained, x3)
    return out.reshape(B, C, H, W)
